# Initial kernel scaffold; baseline (speedup 1.0000x reference)
import jax
import jax.numpy as jnp
from jax.experimental import pallas as pl


def kernel(x, senders, receivers):
    raise NotImplementedError("write your pallas kernel here")


# speedup vs baseline: 3.1236x; 3.1236x over previous
import functools

import jax
import jax.numpy as jnp
from jax import lax
from jax.experimental import pallas as pl
from jax.experimental.pallas import tpu as pltpu
from jax.experimental.pallas import tpu_sc as plsc

NC = 2
NS = 16
NW = NC * NS
L = 16


def _node_dot_sc(x, senders, receivers, *, chunk):
    n_nodes, d_feat = x.shape
    n_edges = senders.shape[0]
    b_per_w = n_edges // NW
    n_chunks = b_per_w // chunk
    n_groups = chunk // L
    mesh = plsc.VectorSubcoreMesh(core_axis_name="c", subcore_axis_name="s")

    @functools.partial(
        pl.kernel,
        mesh=mesh,
        out_type=jax.ShapeDtypeStruct((n_edges,), jnp.float32),
        scratch_types=[
            pltpu.VMEM((chunk,), jnp.int32),
            pltpu.VMEM((chunk,), jnp.int32),
            pltpu.VMEM((chunk, d_feat), jnp.float32),
            pltpu.VMEM((chunk, d_feat), jnp.float32),
            pltpu.VMEM((chunk,), jnp.float32),
            pltpu.SemaphoreType.DMA,
            pltpu.SemaphoreType.DMA,
        ],
    )
    def k(x_hbm, s_hbm, r_hbm, out_hbm, sidx, ridx, srows, rrows, obuf,
          sem_s, sem_r):
        wid = lax.axis_index("s") * NC + lax.axis_index("c")
        base = wid * b_per_w
        rows_iota = jnp.arange(L, dtype=jnp.int32)
        rot_idx = [(rows_iota + sh) % L for sh in (8, 4, 2, 1)]

        def chunk_body(i, carry):
            cb = base + i * chunk
            pltpu.sync_copy(s_hbm.at[pl.ds(cb, chunk)], sidx)
            pltpu.sync_copy(r_hbm.at[pl.ds(cb, chunk)], ridx)
            cs = pltpu.async_copy(x_hbm.at[sidx], srows, sem_s)
            cr = pltpu.async_copy(x_hbm.at[ridx], rrows, sem_r)
            cs.wait()
            cr.wait()

            def group_body(g, carry2):
                eb = g * L
                tot = jnp.zeros((L,), jnp.float32)
                for e in range(L):
                    acc = (srows[eb + e, pl.ds(0, L)]
                           * rrows[eb + e, pl.ds(0, L)])
                    for d in range(1, d_feat // L):
                        acc = acc + (srows[eb + e, pl.ds(d * L, L)]
                                     * rrows[eb + e, pl.ds(d * L, L)])
                    for ridx in rot_idx:
                        acc = acc + acc.at[ridx].get(
                            mode="promise_in_bounds")
                    tot = jnp.where(rows_iota == e, acc, tot)
                obuf[pl.ds(eb, L)] = tot
                return carry2

            lax.fori_loop(0, n_groups, group_body, 0, unroll=False)
            pltpu.sync_copy(obuf, out_hbm.at[pl.ds(cb, chunk)])
            return carry

        lax.fori_loop(0, n_chunks, chunk_body, 0, unroll=False)

    return k(x, senders, receivers)


def kernel(x, senders, receivers):
    return _node_dot_sc(x, senders, receivers, chunk=400)


# speedup vs baseline: 9.9674x; 3.1910x over previous
import functools

import jax
import jax.numpy as jnp
from jax import lax
from jax.experimental import pallas as pl
from jax.experimental.pallas import tpu as pltpu
from jax.experimental.pallas import tpu_sc as plsc

NC = 2
NS = 16
NW = NC * NS
L = 16


def _node_dot_sc(xi, senders, receivers, *, chunk):
    n_nodes, d_words = xi.shape
    n_edges = senders.shape[0]
    b_per_w = n_edges // NW
    n_chunks = b_per_w // chunk
    n_groups = chunk // L
    mesh = plsc.VectorSubcoreMesh(core_axis_name="c", subcore_axis_name="s")

    @functools.partial(
        pl.kernel,
        mesh=mesh,
        compiler_params=pltpu.CompilerParams(use_tc_tiling_on_sc=False),
        out_type=jax.ShapeDtypeStruct((n_edges,), jnp.float32),
        scratch_types=[
            pltpu.VMEM((b_per_w,), jnp.int32),
            pltpu.VMEM((b_per_w,), jnp.int32),
            pltpu.VMEM((chunk, d_words), jnp.int32),
            pltpu.VMEM((chunk, d_words), jnp.int32),
            pltpu.VMEM((chunk, d_words), jnp.int32),
            pltpu.VMEM((chunk, d_words), jnp.int32),
            pltpu.VMEM((chunk,), jnp.float32),
            pltpu.VMEM((chunk,), jnp.float32),
            pltpu.SemaphoreType.DMA,
            pltpu.SemaphoreType.DMA,
            pltpu.SemaphoreType.DMA,
            pltpu.SemaphoreType.DMA,
            pltpu.SemaphoreType.DMA,
            pltpu.SemaphoreType.DMA,
        ],
    )
    def k(x_hbm, s_hbm, r_hbm, out_hbm, sidx, ridx, sbuf0, sbuf1, rbuf0,
          rbuf1, obuf0, obuf1, ss0, ss1, rs0, rs1, os0, os1):
        wid = lax.axis_index("s") * NC + lax.axis_index("c")
        base = wid * b_per_w
        rows_iota = jnp.arange(L, dtype=jnp.int32)
        rot_idx = [(rows_iota + sh) % L for sh in (8, 4, 2, 1)]
        sbufs, rbufs, obufs = (sbuf0, sbuf1), (rbuf0, rbuf1), (obuf0, obuf1)
        ssems, rsems, osems = (ss0, ss1), (rs0, rs1), (os0, os1)
        mask_hi = jnp.int32(-65536)

        pltpu.sync_copy(s_hbm.at[pl.ds(base, b_per_w)], sidx)
        pltpu.sync_copy(r_hbm.at[pl.ds(base, b_per_w)], ridx)

        def fire(g, b):
            pltpu.async_copy(x_hbm.at[sidx.at[pl.ds(g * chunk, chunk)]],
                             sbufs[b], ssems[b])
            pltpu.async_copy(x_hbm.at[ridx.at[pl.ds(g * chunk, chunk)]],
                             rbufs[b], rsems[b])

        def wait_gather(b):
            pltpu.make_async_copy(x_hbm.at[sidx.at[pl.ds(0, chunk)]],
                                  sbufs[b], ssems[b]).wait()
            pltpu.make_async_copy(x_hbm.at[ridx.at[pl.ds(0, chunk)]],
                                  rbufs[b], rsems[b]).wait()

        def wait_out(b):
            pltpu.make_async_copy(obufs[b], out_hbm.at[pl.ds(base, chunk)],
                                  osems[b]).wait()

        def compute_chunk(g, b):
            sb, rb, ob = sbufs[b], rbufs[b], obufs[b]

            def group_body(gr, carry2):
                eb = gr * L
                tot = jnp.zeros((L,), jnp.float32)
                for e in range(L):
                    acc = jnp.zeros((L,), jnp.float32)
                    for d in range(d_words // L):
                        ws = sb[eb + e, pl.ds(d * L, L)]
                        wr = rb[eb + e, pl.ds(d * L, L)]
                        slo = lax.bitcast_convert_type(
                            ws << 16, jnp.float32)
                        shi = lax.bitcast_convert_type(
                            ws & mask_hi, jnp.float32)
                        rlo = lax.bitcast_convert_type(
                            wr << 16, jnp.float32)
                        rhi = lax.bitcast_convert_type(
                            wr & mask_hi, jnp.float32)
                        acc = acc + slo * rlo
                        acc = acc + shi * rhi
                    for ridx_v in rot_idx:
                        acc = acc + acc.at[ridx_v].get(
                            mode="promise_in_bounds")
                    tot = jnp.where(rows_iota == e, acc, tot)
                ob[pl.ds(eb, L)] = tot
                return carry2

            lax.fori_loop(0, n_groups, group_body, 0, unroll=False)
            pltpu.async_copy(
                ob, out_hbm.at[pl.ds(base + g * chunk, chunk)], osems[b])

        fire(0, 0)
        even_end = n_chunks - (n_chunks % 2)

        def outer_body(i, carry):
            for b in range(2):
                g = i * 2 + b
                wait_gather(b)

                @pl.when(g + 1 < n_chunks)
                def _():
                    fire(g + 1, 1 - b)

                @pl.when(g >= 2)
                def _():
                    wait_out(b)

                compute_chunk(g, b)
            return carry

        lax.fori_loop(0, even_end // 2, outer_body, 0, unroll=False)
        if n_chunks % 2 == 1:
            g = n_chunks - 1
            wait_gather(0)
            if n_chunks > 2:
                wait_out(0)
            compute_chunk(g, 0)

        wait_out((n_chunks - 1) % 2)
        if n_chunks > 1:
            wait_out((n_chunks - 2) % 2)

    return k(xi, senders, receivers)


def kernel(x, senders, receivers):
    n_nodes, d_feat = x.shape
    xb = x.astype(jnp.bfloat16).reshape(n_nodes, d_feat // 2, 2)
    xi = lax.bitcast_convert_type(xb, jnp.int32)
    return _node_dot_sc(xi, senders, receivers, chunk=400)


# speedup vs baseline: 11.1502x; 1.1187x over previous
import functools

import jax
import jax.numpy as jnp
from jax import lax
from jax.experimental import pallas as pl
from jax.experimental.pallas import tpu as pltpu
from jax.experimental.pallas import tpu_sc as plsc

NC = 2
NS = 16
NW = NC * NS
L = 16


def _node_dot_sc(xi, senders, receivers, *, chunk):
    n_nodes, d_words = xi.shape
    n_edges = senders.shape[0]
    b_per_w = n_edges // NW
    n_chunks = b_per_w // chunk
    n_groups = chunk // L
    mesh = plsc.VectorSubcoreMesh(core_axis_name="c", subcore_axis_name="s")

    @functools.partial(
        pl.kernel,
        mesh=mesh,
        compiler_params=pltpu.CompilerParams(use_tc_tiling_on_sc=False),
        out_type=jax.ShapeDtypeStruct((n_edges,), jnp.float32),
        scratch_types=[
            pltpu.VMEM((b_per_w,), jnp.int32),
            pltpu.VMEM((b_per_w,), jnp.int32),
            pltpu.VMEM((chunk, d_words), jnp.int32),
            pltpu.VMEM((chunk, d_words), jnp.int32),
            pltpu.VMEM((chunk, d_words), jnp.int32),
            pltpu.VMEM((chunk, d_words), jnp.int32),
            pltpu.VMEM((chunk,), jnp.float32),
            pltpu.VMEM((chunk,), jnp.float32),
            pltpu.SemaphoreType.DMA,
            pltpu.SemaphoreType.DMA,
            pltpu.SemaphoreType.DMA,
            pltpu.SemaphoreType.DMA,
            pltpu.SemaphoreType.DMA,
            pltpu.SemaphoreType.DMA,
        ],
    )
    def k(x_hbm, s_hbm, r_hbm, out_hbm, sidx, ridx, sbuf0, sbuf1, rbuf0,
          rbuf1, obuf0, obuf1, ss0, ss1, rs0, rs1, os0, os1):
        wid = lax.axis_index("s") * NC + lax.axis_index("c")
        base = wid * b_per_w
        rows_iota = jnp.arange(L, dtype=jnp.int32)
        rot_idx = [(rows_iota + sh) % L for sh in (8, 4, 2, 1)]
        sbufs, rbufs, obufs = (sbuf0, sbuf1), (rbuf0, rbuf1), (obuf0, obuf1)
        ssems, rsems, osems = (ss0, ss1), (rs0, rs1), (os0, os1)
        mask_hi = jnp.int32(-65536)

        pltpu.sync_copy(s_hbm.at[pl.ds(base, b_per_w)], sidx)
        pltpu.sync_copy(r_hbm.at[pl.ds(base, b_per_w)], ridx)

        def fire(g, b):
            pltpu.async_copy(x_hbm.at[sidx.at[pl.ds(g * chunk, chunk)]],
                             sbufs[b], ssems[b])
            pltpu.async_copy(x_hbm.at[ridx.at[pl.ds(g * chunk, chunk)]],
                             rbufs[b], rsems[b])

        def wait_gather(b):
            pltpu.make_async_copy(x_hbm.at[sidx.at[pl.ds(0, chunk)]],
                                  sbufs[b], ssems[b]).wait()
            pltpu.make_async_copy(x_hbm.at[ridx.at[pl.ds(0, chunk)]],
                                  rbufs[b], rsems[b]).wait()

        def wait_out(b):
            pltpu.make_async_copy(obufs[b], out_hbm.at[pl.ds(base, chunk)],
                                  osems[b]).wait()

        def compute_chunk(g, b):
            sb, rb, ob = sbufs[b], rbufs[b], obufs[b]

            def group_body(gr, carry2):
                eb = gr * L
                tot = jnp.zeros((L,), jnp.float32)
                for e in range(L):
                    acc = jnp.zeros((L,), jnp.float32)
                    for d in range(d_words // L):
                        ws = sb[eb + e, pl.ds(d * L, L)]
                        wr = rb[eb + e, pl.ds(d * L, L)]
                        slo = lax.bitcast_convert_type(
                            ws << 16, jnp.float32)
                        rlo = lax.bitcast_convert_type(
                            wr << 16, jnp.float32)
                        shi = lax.bitcast_convert_type(ws, jnp.float32)
                        rhi = lax.bitcast_convert_type(wr, jnp.float32)
                        acc = acc + slo * rlo
                        acc = acc + shi * rhi
                    for ridx_v in rot_idx:
                        acc = acc + acc.at[ridx_v].get(
                            mode="promise_in_bounds")
                    tot = jnp.where(rows_iota == e, acc, tot)
                ob[pl.ds(eb, L)] = tot
                return carry2

            lax.fori_loop(0, n_groups, group_body, 0, unroll=False)
            pltpu.async_copy(
                ob, out_hbm.at[pl.ds(base + g * chunk, chunk)], osems[b])

        fire(0, 0)
        even_end = n_chunks - (n_chunks % 2)

        def outer_body(i, carry):
            for b in range(2):
                g = i * 2 + b
                wait_gather(b)

                @pl.when(g + 1 < n_chunks)
                def _():
                    fire(g + 1, 1 - b)

                @pl.when(g >= 2)
                def _():
                    wait_out(b)

                compute_chunk(g, b)
            return carry

        lax.fori_loop(0, even_end // 2, outer_body, 0, unroll=False)
        if n_chunks % 2 == 1:
            g = n_chunks - 1
            wait_gather(0)
            if n_chunks > 2:
                wait_out(0)
            compute_chunk(g, 0)

        wait_out((n_chunks - 1) % 2)
        if n_chunks > 1:
            wait_out((n_chunks - 2) % 2)

    return k(xi, senders, receivers)


def kernel(x, senders, receivers):
    n_nodes, d_feat = x.shape
    xb = x.astype(jnp.bfloat16).reshape(n_nodes, d_feat // 2, 2)
    xi = lax.bitcast_convert_type(xb, jnp.int32)
    return _node_dot_sc(xi, senders, receivers, chunk=400)
